# baseline (speedup 1.0000x reference)
import functools

import jax
import jax.numpy as jnp
from jax import lax
from jax.experimental import pallas as pl
from jax.experimental.pallas import tpu as pltpu
from jax.experimental.pallas import tpu_sc as plsc

B = 1024
Q = 100
C = 92
K = 4
L = 16
QG = 7
NW = 32
BPW = B // NW
CH = 4
NCH = BPW // CH

_mesh = plsc.VectorSubcoreMesh(core_axis_name="c", subcore_axis_name="s")


@functools.partial(
    pl.kernel,
    mesh=_mesh,
    compiler_params=pltpu.CompilerParams(needs_layout_passes=False,
                                         use_tc_tiling_on_sc=False),
    out_type=jax.ShapeDtypeStruct((B * Q,), jnp.float32),
    scratch_types=[
        pltpu.VMEM((CH, Q, C), jnp.float32),
        pltpu.VMEM((CH, Q, C), jnp.float32),
        pltpu.VMEM((BPW * Q + L,), jnp.float32),
        pltpu.SemaphoreType.DMA,
        pltpu.SemaphoreType.DMA,
    ],
)
def _mask_kernel(logits_hbm, out_hbm, buf0, buf1, maskbuf, sem0, sem1):
    wid = lax.axis_index("s") * 2 + lax.axis_index("c")
    b0 = wid * BPW

    lane = jnp.arange(L, dtype=jnp.int32)
    zero_i = jnp.zeros((L,), jnp.int32)
    lanes_g = [g * L + lane for g in range(QG)]
    qv = [jnp.minimum(lanes_g[g], Q - 1) for g in range(QG)]

    def compute(buf, chbase, j):
        boff = chbase + j * Q
        jb = zero_i + j

        def cstep(i, ms):
            cc = zero_i + (1 + i * 3)
            out = []
            for g in range(QG):
                v0 = plsc.load_gather(buf, [jb, qv[g], cc])
                v1 = plsc.load_gather(buf, [jb, qv[g], cc + 1])
                v2 = plsc.load_gather(buf, [jb, qv[g], cc + 2])
                out.append(jnp.maximum(jnp.maximum(ms[g], v0),
                                       jnp.maximum(v1, v2)))
            return tuple(out)

        m0 = tuple(plsc.load_gather(buf, [jb, qv[g], zero_i])
                   for g in range(QG))
        ms = lax.fori_loop(0, 30, cstep, m0)

        masks = []
        acc = jnp.zeros((L,), jnp.float32)
        for g in range(QG):
            l91 = plsc.load_gather(buf, [jb, qv[g], zero_i + (C - 1)])
            m1 = jnp.where(ms[g] >= l91, 1.0, 0.0).astype(jnp.float32)
            masks.append(m1)
            acc = acc + m1
        for g in range(QG):
            maskbuf[pl.ds(boff + g * L, L)] = masks[g]

        no_det = jnp.sum(acc) < 1.0

        def fallback():
            def sstep(c, ss):
                cc = zero_i + c
                return tuple(
                    ss[g] + jnp.exp(plsc.load_gather(buf, [jb, qv[g], cc])
                                    - ms[g])
                    for g in range(QG))
            ss = lax.fori_loop(0, C - 1, sstep,
                               tuple(jnp.zeros((L,), jnp.float32)
                                     for _ in range(QG)))
            scores = tuple(
                jnp.where(lanes_g[g] < Q, 1.0 / ss[g], -1.0)
                for g in range(QG))
            zeros = tuple(jnp.zeros((L,), jnp.float32) for _ in range(QG))

            def kstep(_, carry):
                sc, mk = carry
                mv = sc[0]
                for g in range(1, QG):
                    mv = jnp.maximum(mv, sc[g])
                gmax = jnp.max(mv)
                big = jnp.full((L,), 10**9, jnp.int32)
                bv = big
                for g in range(QG):
                    bv = jnp.minimum(
                        bv, jnp.where(sc[g] == gmax, lanes_g[g], big))
                gidx = jnp.min(bv)
                sc = tuple(jnp.where(lanes_g[g] == gidx, -1.0, sc[g])
                           for g in range(QG))
                mk = tuple(jnp.where(lanes_g[g] == gidx, 1.0, mk[g])
                           for g in range(QG))
                return sc, mk

            _, mk = lax.fori_loop(0, K, kstep, (scores, zeros))
            for g in range(QG):
                maskbuf[pl.ds(boff + g * L, L)] = mk[g]

        pl.when(no_det)(fallback)

    def compute_chunk(buf, ch):
        chbase = ch * (CH * Q)
        lax.fori_loop(0, CH, lambda j, _: (compute(buf, chbase, j), 0)[1], 0)

    def chunk_src(ch):
        return logits_hbm.at[pl.ds(b0 + ch * CH, CH)]

    pltpu.async_copy(chunk_src(0), buf0, sem0)
    pltpu.async_copy(chunk_src(1), buf1, sem1)

    def body(i, _):
        ch = 2 * i
        pltpu.make_async_copy(chunk_src(ch), buf0, sem0).wait()
        compute_chunk(buf0, ch)
        pltpu.async_copy(chunk_src(ch + 2), buf0, sem0)
        pltpu.make_async_copy(chunk_src(ch + 1), buf1, sem1).wait()
        compute_chunk(buf1, ch + 1)
        pltpu.async_copy(chunk_src(ch + 3), buf1, sem1)
        return 0

    lax.fori_loop(0, NCH // 2 - 1, body, 0)

    ch = NCH - 2
    pltpu.make_async_copy(chunk_src(ch), buf0, sem0).wait()
    compute_chunk(buf0, ch)
    pltpu.make_async_copy(chunk_src(ch + 1), buf1, sem1).wait()
    compute_chunk(buf1, ch + 1)

    pltpu.sync_copy(maskbuf.at[pl.ds(0, BPW * Q)],
                    out_hbm.at[pl.ds(b0 * Q, BPW * Q)])


def kernel(input_modal, detr_features, detr_logits):
    mask_flat = _mask_kernel(detr_logits)
    return detr_features, mask_flat.reshape(B, Q)


# speedup vs baseline: 2.1360x; 2.1360x over previous
import functools

import jax
import jax.numpy as jnp
from jax import lax
from jax.experimental import pallas as pl
from jax.experimental.pallas import tpu as pltpu
from jax.experimental.pallas import tpu_sc as plsc

B = 1024
Q = 100
C = 92
K = 4
L = 16
QG = 7
NW = 32
BPW = B // NW
NEG = -3.0e38

_mesh = plsc.VectorSubcoreMesh(core_axis_name="c", subcore_axis_name="s")


@functools.partial(
    pl.kernel,
    mesh=_mesh,
    compiler_params=pltpu.CompilerParams(needs_layout_passes=False),
    out_type=jax.ShapeDtypeStruct((B * Q,), jnp.float32),
    scratch_types=[
        pltpu.VMEM((Q, C), jnp.float32),
        pltpu.VMEM((Q, C), jnp.float32),
        pltpu.VMEM((BPW * Q + L,), jnp.float32),
        pltpu.VMEM((QG * L,), jnp.float32),
        pltpu.SemaphoreType.DMA,
        pltpu.SemaphoreType.DMA,
    ],
)
def _mask_kernel(logits_hbm, out_hbm, buf0, buf1, maskbuf, scorebuf,
                 sem0, sem1):
    wid = lax.axis_index("s") * 2 + lax.axis_index("c")
    b0 = wid * BPW

    lane = jnp.arange(L, dtype=jnp.int32)
    lanes_g = [g * L + lane for g in range(QG)]
    tail_max = lane < 15
    tail_exp = (lane >= 4) & (lane < 15)

    def row_members(buf, q):
        v5 = buf[q, pl.ds(C - L, L)]
        v01 = jnp.maximum(buf[q, pl.ds(0, L)], buf[q, pl.ds(L, L)])
        v23 = jnp.maximum(buf[q, pl.ds(2 * L, L)], buf[q, pl.ds(3 * L, L)])
        v4 = buf[q, pl.ds(4 * L, L)]
        v5m = jnp.where(tail_max, v5, NEG)
        mv = jnp.maximum(jnp.maximum(v01, v23), jnp.maximum(v4, v5m))
        return mv, v5[L - 1]

    def compute(buf, boff):
        def gstep(g, acc):
            def qstep(j, mvec):
                q = jnp.minimum(g * L + j, Q - 1)
                mv, l91 = row_members(buf, q)
                m1 = jnp.where(jnp.max(mv) >= l91, 1.0, 0.0)
                return jnp.where(lane == j, m1, mvec)

            mvec = lax.fori_loop(0, L, qstep, jnp.zeros((L,), jnp.float32))
            maskbuf[pl.ds(boff + g * L, L)] = mvec
            return acc + mvec

        acc = lax.fori_loop(0, QG, gstep, jnp.zeros((L,), jnp.float32))
        no_det = jnp.sum(acc) < 1.0

        def fallback():
            def sgroup(g, _):
                def srow(j, svec):
                    q = jnp.minimum(g * L + j, Q - 1)
                    mv, _ = row_members(buf, q)
                    m = jnp.max(mv)
                    e = (jnp.exp(buf[q, pl.ds(0, L)] - m)
                         + jnp.exp(buf[q, pl.ds(L, L)] - m)
                         + jnp.exp(buf[q, pl.ds(2 * L, L)] - m)
                         + jnp.exp(buf[q, pl.ds(3 * L, L)] - m)
                         + jnp.exp(buf[q, pl.ds(4 * L, L)] - m))
                    e5 = jnp.where(tail_exp,
                                   jnp.exp(buf[q, pl.ds(C - L, L)] - m), 0.0)
                    s = jnp.sum(e + e5)
                    recip = 1.0 / (jnp.zeros((L,), jnp.float32) + s)
                    return jnp.where(lane == j, recip, svec)

                svec = lax.fori_loop(
                    0, L, srow, jnp.full((L,), -1.0, jnp.float32))
                svec = jnp.where(g * L + lane < Q, svec, -1.0)
                scorebuf[pl.ds(g * L, L)] = svec
                return 0

            lax.fori_loop(0, QG, sgroup, 0)

            scores = tuple(scorebuf[pl.ds(g * L, L)] for g in range(QG))
            zeros = tuple(jnp.zeros((L,), jnp.float32) for _ in range(QG))

            def kstep(_, carry):
                sc, mk = carry
                mv = sc[0]
                for g in range(1, QG):
                    mv = jnp.maximum(mv, sc[g])
                gmax = jnp.max(mv)
                big = jnp.full((L,), 10**9, jnp.int32)
                bv = big
                for g in range(QG):
                    bv = jnp.minimum(
                        bv, jnp.where(sc[g] == gmax, lanes_g[g], big))
                gidx = jnp.min(bv)
                sc = tuple(jnp.where(lanes_g[g] == gidx, -1.0, sc[g])
                           for g in range(QG))
                mk = tuple(jnp.where(lanes_g[g] == gidx, 1.0, mk[g])
                           for g in range(QG))
                return sc, mk

            _, mk = lax.fori_loop(0, K, kstep, (scores, zeros))
            for g in range(QG):
                maskbuf[pl.ds(boff + g * L, L)] = mk[g]

        pl.when(no_det)(fallback)

    pltpu.async_copy(logits_hbm.at[b0], buf0, sem0)
    pltpu.async_copy(logits_hbm.at[b0 + 1], buf1, sem1)

    def body(i, _):
        bl = 2 * i
        pltpu.make_async_copy(logits_hbm.at[b0 + bl], buf0, sem0).wait()
        compute(buf0, bl * Q)
        pltpu.async_copy(logits_hbm.at[b0 + bl + 2], buf0, sem0)
        pltpu.make_async_copy(logits_hbm.at[b0 + bl + 1], buf1, sem1).wait()
        compute(buf1, (bl + 1) * Q)
        pltpu.async_copy(logits_hbm.at[b0 + bl + 3], buf1, sem1)
        return 0

    lax.fori_loop(0, BPW // 2 - 1, body, 0)

    bl = BPW - 2
    pltpu.make_async_copy(logits_hbm.at[b0 + bl], buf0, sem0).wait()
    compute(buf0, bl * Q)
    pltpu.make_async_copy(logits_hbm.at[b0 + bl + 1], buf1, sem1).wait()
    compute(buf1, (bl + 1) * Q)

    pltpu.sync_copy(maskbuf.at[pl.ds(0, BPW * Q)],
                    out_hbm.at[pl.ds(b0 * Q, BPW * Q)])


def kernel(input_modal, detr_features, detr_logits):
    mask_flat = _mask_kernel(detr_logits)
    return detr_features, mask_flat.reshape(B, Q)
